# baseline (speedup 1.0000x reference)
import functools

import jax
import jax.numpy as jnp
from jax import lax
from jax.experimental import pallas as pl
from jax.experimental.pallas import tpu as pltpu
from jax.experimental.pallas import tpu_sc as plsc


@functools.cache
def _make_gather(num_rows: int, d: int, batch: int):
    info = plsc.get_sparse_core_info()
    nw = info.num_cores * info.num_subcores
    assert batch % nw == 0
    b_per_w = batch // nw
    assert (b_per_w % 8) == 0 and (d % info.num_lanes) == 0
    nc = info.num_cores
    mesh = plsc.VectorSubcoreMesh(core_axis_name="c", subcore_axis_name="s")

    @functools.partial(
        pl.kernel,
        mesh=mesh,
        out_type=jax.ShapeDtypeStruct((batch, d), jnp.float32),
        scratch_types=[
            pltpu.VMEM((b_per_w,), jnp.int32),
            pltpu.VMEM((b_per_w, d), jnp.float32),
            pltpu.SemaphoreType.DMA,
        ],
    )
    def gather_kernel(table_hbm, idx_hbm, out_hbm, idx_v, rows_v, sem):
        wid = lax.axis_index("s") * nc + lax.axis_index("c")
        base = wid * b_per_w
        pltpu.sync_copy(idx_hbm.at[pl.ds(base, b_per_w)], idx_v)
        pltpu.async_copy(table_hbm.at[idx_v], rows_v, sem).wait()
        pltpu.sync_copy(rows_v, out_hbm.at[pl.ds(base, b_per_w)])

    return gather_kernel


def kernel(class_number, embedding):
    num_rows, pts, ch = embedding.shape
    batch = class_number.shape[0]
    d = pts * ch
    table = embedding.reshape(num_rows, d)
    idx = class_number.astype(jnp.int32)
    out = _make_gather(num_rows, d, batch)(table, idx)
    return out.reshape(batch, pts, ch)


# speedup vs baseline: 20.6824x; 20.6824x over previous
import functools

import jax
import jax.numpy as jnp
from jax import lax
from jax.experimental import pallas as pl
from jax.experimental.pallas import tpu as pltpu
from jax.experimental.pallas import tpu_sc as plsc

_CHUNK = 128


@functools.cache
def _make_gather(num_rows: int, lanes: int, n_chunks: int):
    info = plsc.get_sparse_core_info()
    nw = info.num_cores * info.num_subcores
    nc = info.num_cores
    assert n_chunks % nw == 0
    c_per_w = n_chunks // nw
    mesh = plsc.VectorSubcoreMesh(core_axis_name="c", subcore_axis_name="s")

    @functools.partial(
        pl.kernel,
        mesh=mesh,
        out_type=jax.ShapeDtypeStruct((nw, c_per_w, _CHUNK, lanes), jnp.float32),
        scratch_types=[
            pltpu.VMEM((c_per_w, _CHUNK), jnp.int32),
            pltpu.VMEM((c_per_w, _CHUNK, lanes), jnp.float32),
            pltpu.SemaphoreType.DMA,
        ],
    )
    def gather_kernel(table_hbm, idx_hbm, out_hbm, idx_v, rows_v, sem):
        wid = lax.axis_index("s") * nc + lax.axis_index("c")
        pltpu.sync_copy(idx_hbm.at[wid], idx_v)
        copies = [
            pltpu.async_copy(table_hbm.at[idx_v.at[r]], rows_v.at[r], sem)
            for r in range(c_per_w)
        ]
        for c in copies:
            c.wait()
        pltpu.sync_copy(rows_v, out_hbm.at[wid])

    return gather_kernel


def kernel(class_number, embedding):
    num_classes, pts, ch = embedding.shape
    batch = class_number.shape[0]
    table = jnp.moveaxis(embedding, 2, 0).reshape(num_classes * ch, pts)
    idx = class_number.astype(jnp.int32)
    idx3 = (idx[None, :] + (jnp.arange(ch, dtype=jnp.int32) * num_classes)[:, None])
    n_chunks = (ch * batch) // _CHUNK
    nw = 32
    idx_m = idx3.reshape(nw, n_chunks // nw, _CHUNK)
    out = _make_gather(num_classes * ch, pts, n_chunks)(table, idx_m)
    return jnp.moveaxis(out.reshape(ch, batch, pts), 0, 2)
